# baseline (device time: 61598 ns/iter reference)
import jax
import jax.numpy as jnp
from jax import lax
from jax.experimental import pallas as pl
from jax.experimental.pallas import tpu as pltpu

N_DEV = 8
N_HOPS = N_DEV - 1


def kernel(x, dy):
    k, m = x.shape
    _, n = dy.shape
    m_out = m // N_DEV

    def body(x_ref, dy_ref, out_ref, p_ref, comm_ref, send_sems, recv_sems):
        my = lax.axis_index("i")
        left = lax.rem(my + N_DEV - 1, N_DEV)
        right = lax.rem(my + 1, N_DEV)

        barrier = pltpu.get_barrier_semaphore()
        for nbr in (left, right):
            pl.semaphore_signal(
                barrier, inc=1, device_id=(nbr,),
                device_id_type=pl.DeviceIdType.MESH,
            )
        pl.semaphore_wait(barrier, 2)

        xb = x_ref[...].astype(jnp.bfloat16)
        db = dy_ref[...].astype(jnp.bfloat16)
        p_ref[...] = lax.dot_general(
            xb, db, (((0,), (0,)), ((), ())),
            preferred_element_type=jnp.float32,
        )

        c0 = lax.rem(my + N_DEV - 1, N_DEV)
        comm_ref[0] = p_ref[pl.ds(c0 * m_out, m_out), :]

        for h in range(N_HOPS):
            rdma = pltpu.make_async_remote_copy(
                src_ref=comm_ref.at[h],
                dst_ref=comm_ref.at[h + 1],
                send_sem=send_sems.at[h],
                recv_sem=recv_sems.at[h],
                device_id=(right,),
                device_id_type=pl.DeviceIdType.MESH,
            )
            rdma.start()
            rdma.wait()
            c = lax.rem(my + 2 * N_DEV - 2 - h, N_DEV)
            comm_ref[h + 1] = (
                comm_ref[h + 1] + p_ref[pl.ds(c * m_out, m_out), :]
            )

        out_ref[...] = comm_ref[N_HOPS]

    return pl.pallas_call(
        body,
        out_shape=jax.ShapeDtypeStruct((m_out, n), jnp.float32),
        in_specs=[
            pl.BlockSpec(memory_space=pltpu.VMEM),
            pl.BlockSpec(memory_space=pltpu.VMEM),
        ],
        out_specs=pl.BlockSpec(memory_space=pltpu.VMEM),
        scratch_shapes=[
            pltpu.VMEM((m, n), jnp.float32),
            pltpu.VMEM((N_DEV, m_out, n), jnp.float32),
            pltpu.SemaphoreType.DMA((N_HOPS,)),
            pltpu.SemaphoreType.DMA((N_HOPS,)),
        ],
        compiler_params=pltpu.CompilerParams(collective_id=0),
    )(x, dy)


# device time: 24154 ns/iter; 2.5502x vs baseline; 2.5502x over previous
import jax
import jax.numpy as jnp
from jax import lax
from jax.experimental import pallas as pl
from jax.experimental.pallas import tpu as pltpu

N_DEV = 8


def kernel(x, dy):
    k, m = x.shape
    _, n = dy.shape
    m_out = m // N_DEV

    def body(x_ref, dy_ref, out_ref, pc_ref, recv_ref, send_sems, recv_sems):
        my = lax.axis_index("i")

        barrier = pltpu.get_barrier_semaphore()
        for d in range(N_DEV):
            @pl.when(my != d)
            def _():
                pl.semaphore_signal(
                    barrier, inc=1, device_id=(d,),
                    device_id_type=pl.DeviceIdType.MESH,
                )
        pl.semaphore_wait(barrier, N_DEV - 1)

        xb = x_ref[...].astype(jnp.bfloat16)
        db = dy_ref[...].astype(jnp.bfloat16)
        p = lax.dot_general(
            xb, db, (((0,), (0,)), ((), ())),
            preferred_element_type=jnp.float32,
        )
        pc_ref[...] = p.astype(jnp.bfloat16).reshape(N_DEV, m_out, n)

        sends = []
        for c in range(N_DEV):
            rdma = pltpu.make_async_remote_copy(
                src_ref=pc_ref.at[c],
                dst_ref=recv_ref.at[my],
                send_sem=send_sems.at[c],
                recv_sem=recv_sems.at[my],
                device_id=(c,),
                device_id_type=pl.DeviceIdType.MESH,
            )
            sends.append(rdma)

            @pl.when(my != c)
            def _():
                rdma.start()

        recv_ref[pl.ds(my, 1)] = pc_ref[pl.ds(my, 1)]

        for s in range(N_DEV):
            recv = pltpu.make_async_remote_copy(
                src_ref=recv_ref.at[s],
                dst_ref=recv_ref.at[s],
                send_sem=send_sems.at[s],
                recv_sem=recv_sems.at[s],
                device_id=(s,),
                device_id_type=pl.DeviceIdType.MESH,
            )

            @pl.when(my != s)
            def _():
                recv.wait_recv()

        out_ref[...] = jnp.sum(recv_ref[...].astype(jnp.float32), axis=0)

        for c in range(N_DEV):
            @pl.when(my != c)
            def _():
                sends[c].wait_send()

    return pl.pallas_call(
        body,
        out_shape=jax.ShapeDtypeStruct((m_out, n), jnp.float32),
        in_specs=[
            pl.BlockSpec(memory_space=pltpu.VMEM),
            pl.BlockSpec(memory_space=pltpu.VMEM),
        ],
        out_specs=pl.BlockSpec(memory_space=pltpu.VMEM),
        scratch_shapes=[
            pltpu.VMEM((N_DEV, m_out, n), jnp.bfloat16),
            pltpu.VMEM((N_DEV, m_out, n), jnp.bfloat16),
            pltpu.SemaphoreType.DMA((N_DEV,)),
            pltpu.SemaphoreType.DMA((N_DEV,)),
        ],
        compiler_params=pltpu.CompilerParams(collective_id=0),
    )(x, dy)
